# baseline (speedup 1.0000x reference)
import functools
import math

import jax
import jax.numpy as jnp
from jax import lax
from jax.experimental import pallas as pl
from jax.experimental.pallas import tpu as pltpu
from jax.experimental.pallas import tpu_sc as plsc

_LANES = 128
_NW = 32

_GDN = lax.GatherDimensionNumbers(
    offset_dims=(), collapsed_slice_dims=(0,), start_index_map=(0,))


def _bcast_lane(vec, e):
    return lax.gather(vec, jnp.full((16, 1), e, jnp.int32), _GDN, (1,),
                      mode=lax.GatherScatterMode.PROMISE_IN_BOUNDS)


def _make_deg_kernel(n_chunks, npt):
    mesh = plsc.VectorSubcoreMesh(core_axis_name="c", subcore_axis_name="s")

    @functools.partial(
        pl.kernel,
        out_type=jax.ShapeDtypeStruct((2, 16, npt), jnp.float32),
        mesh=mesh,
        scratch_types=[
            pltpu.VMEM((n_chunks, _LANES), jnp.int32),
            pltpu.VMEM((n_chunks, _LANES), jnp.float32),
            pltpu.VMEM((npt,), jnp.float32),
            pltpu.VMEM_SHARED((16 * npt,), jnp.float32),
            pltpu.SemaphoreType.DMA,
        ],
    )
    def deg_kernel(col_hbm, ew_hbm, out_hbm, colv, ewv, zbuf, deg_sh, sem):
        c = lax.axis_index("c")
        s = lax.axis_index("s")
        wid = c * 16 + s

        def zero_body(i, carry):
            zbuf[pl.ds(i * 16, 16)] = jnp.zeros((16,), jnp.float32)
            return carry

        lax.fori_loop(0, npt // 16, zero_body, 0)

        pltpu.sync_copy(zbuf, deg_sh.at[pl.ds(s * npt, npt)])
        plsc.subcore_barrier()
        pltpu.sync_copy(col_hbm.at[wid], colv)
        pltpu.sync_copy(ew_hbm.at[wid], ewv)

        def body(j, carry):
            pltpu.sync_copy(ewv.at[j], deg_sh.at[colv.at[j]], add=True)
            return carry

        lax.fori_loop(0, n_chunks, body, 0)
        plsc.subcore_barrier()
        pltpu.sync_copy(deg_sh.at[pl.ds(s * npt, npt)], zbuf)
        pltpu.sync_copy(zbuf, out_hbm.at[c, s])

    return deg_kernel


_CH = 128
_GC = 8


def _make_edge_kernel(n_pad, n_g):
    nps = n_pad // 16
    wb = 64
    n_wb = nps // wb
    mesh = plsc.VectorSubcoreMesh(core_axis_name="c", subcore_axis_name="s")

    @functools.partial(
        pl.kernel,
        out_type=jax.ShapeDtypeStruct((2, 16, n_wb, wb, 128), jnp.float32),
        mesh=mesh,
        scratch_types=[
            pltpu.VMEM((2 * _GC, _CH), jnp.int32),
            pltpu.VMEM((4 * _GC, 64), jnp.int32),
            pltpu.VMEM((2 * 8 * _GC, 16), jnp.float32),
            pltpu.VMEM((_CH, 64), jnp.int32),
            pltpu.VMEM((64, 128), jnp.float32),
            pltpu.VMEM_SHARED((n_pad, 128), jnp.float32),
            pltpu.SemaphoreType.DMA,
            pltpu.SemaphoreType.DMA,
            pltpu.SemaphoreType.DMA,
            pltpu.SemaphoreType.DMA,
        ],
        compiler_params=pltpu.CompilerParams(use_tc_tiling_on_sc=False),
    )
    def edge_kernel(gidx_hbm, cidx_hbm, ew_hbm, xw2p_hbm, out_hbm,
                    gidxs, cidxs, ews, pbuf, fbuf, acc_sh, sl0, sl1, sg, ss):
        sl = (sl0, sl1)
        c = lax.axis_index("c")
        s = lax.axis_index("s")
        wid = c * 16 + s

        def zero_body(t, carry):
            fbuf[t >> 3, pl.ds((t & 7) * 16, 16)] = jnp.zeros(
                (16,), jnp.float32)
            return carry

        lax.fori_loop(0, wb * 8, zero_body, 0)

        def zcopy(k, carry):
            pltpu.sync_copy(fbuf, acc_sh.at[pl.ds(s * nps + k * wb, wb)])
            return carry

        lax.fori_loop(0, n_wb, zcopy, 0)
        plsc.subcore_barrier()

        pltpu.async_copy(gidx_hbm.at[wid, 0], gidxs.at[pl.ds(0, _GC)], sl[0])
        pltpu.async_copy(cidx_hbm.at[wid, 0],
                         cidxs.at[pl.ds(0, 2 * _GC)], sl[0])
        pltpu.async_copy(ew_hbm.at[wid, 0], ews.at[pl.ds(0, 8 * _GC)], sl[0])

        def slotstage(gx, slot):
            other = 1 - slot
            g = 2 * gx + slot
            gb = slot * _GC
            cb = slot * 2 * _GC
            eb = slot * 8 * _GC

            @pl.when(g + 1 < n_g)
            def _():
                pltpu.async_copy(gidx_hbm.at[wid, g + 1],
                                 gidxs.at[pl.ds(other * _GC, _GC)], sl[other])
                pltpu.async_copy(
                    cidx_hbm.at[wid, g + 1],
                    cidxs.at[pl.ds(other * 2 * _GC, 2 * _GC)], sl[other])
                pltpu.async_copy(ew_hbm.at[wid, g + 1],
                                 ews.at[pl.ds(other * 8 * _GC, 8 * _GC)],
                                 sl[other])

            pltpu.make_async_copy(
                gidx_hbm.at[wid, g], gidxs.at[pl.ds(gb, _GC)], sl[slot]).wait()
            pltpu.make_async_copy(
                cidx_hbm.at[wid, g], cidxs.at[pl.ds(cb, 2 * _GC)],
                sl[slot]).wait()
            pltpu.make_async_copy(
                ew_hbm.at[wid, g], ews.at[pl.ds(eb, 8 * _GC)], sl[slot]).wait()

            def chunk(jj, carry):
                pltpu.async_copy(xw2p_hbm.at[gidxs.at[gb + jj]], pbuf,
                                 sg).wait()

                for h in range(2):
                    def scale(g4, carry2):
                        wvec = ews[eb + jj * 8 + h * 4 + g4, pl.ds(0, 16)]
                        for e in range(16):
                            w = _bcast_lane(wvec, e)
                            rr = h * 64 + g4 * 16 + e
                            fr = g4 * 16 + e
                            for kk in range(4):
                                pv = pbuf[rr, pl.ds(kk * 16, 16)]
                                a = lax.bitcast_convert_type(
                                    lax.shift_left(pv, 16), jnp.float32)
                                b = lax.bitcast_convert_type(
                                    jnp.bitwise_and(pv, jnp.int32(-65536)),
                                    jnp.float32)
                                fbuf[fr, pl.ds(kk * 32, 16)] = a * w
                                fbuf[fr, pl.ds(kk * 32 + 16, 16)] = b * w
                        return carry2

                    lax.fori_loop(0, 4, scale, 0)
                    pltpu.async_copy(fbuf,
                                     acc_sh.at[cidxs.at[cb + jj * 2 + h]],
                                     ss, add=True).wait()
                return carry

            lax.fori_loop(0, _GC, chunk, 0)

        def group(gx, carry):
            slotstage(gx, 0)
            slotstage(gx, 1)
            return carry

        lax.fori_loop(0, n_g // 2, group, 0)
        plsc.subcore_barrier()

        def wback(k, carry):
            base = s * nps + k * wb
            pltpu.sync_copy(acc_sh.at[pl.ds(base, wb)], fbuf)
            pltpu.sync_copy(fbuf, out_hbm.at[c, s, k])
            return carry

        lax.fori_loop(0, n_wb, wback, 0)

    return edge_kernel


_OUTER_DN = (((0,), (0,)), ((), ()))


def _expand_rows(v_1r, width):
    ones = jnp.ones((1, width), jnp.float32)
    return lax.dot_general(v_1r, ones, _OUTER_DN,
                           preferred_element_type=jnp.float32)


def _tc_xw2_body(x_ref, w_ref, dm_ref, xw2_ref, dinv_ref):
    deg_row = dm_ref[0, 0] + dm_ref[1, 0] + 1.0
    deg = _expand_rows(deg_row, 128)
    dinv = lax.rsqrt(deg)
    xw = jnp.dot(x_ref[...], w_ref[...], preferred_element_type=jnp.float32)
    xw2_ref[...] = xw * dinv
    dinv_ref[...] = dinv


def _make_final_body(nblk, nb):
    def body(pre_ref, xw2_ref, dinv_ref, batch_ref, bin_ref,
             w1a_ref, w1b_ref, b1_ref, w2_ref, b2_ref, out_ref,
             acc_add, acc_max):
        i = pl.program_id(0)

        @pl.when(i == 0)
        def _():
            acc_add[...] = jnp.zeros_like(acc_add)
            acc_max[...] = jnp.zeros_like(acc_max)

        r = pre_ref.shape[1]
        pre = pre_ref[0] + pre_ref[1]
        h = jnp.maximum((pre + xw2_ref[...]) * dinv_ref[...] + bin_ref[...],
                        0.0)
        bv = batch_ref[0]
        mask = (jnp.broadcast_to(bv, (nb, r))
                == lax.broadcasted_iota(jnp.int32, (nb, r), 0))
        maskf = mask.astype(jnp.float32)
        acc_add[...] += jnp.dot(maskf, h,
                                preferred_element_type=jnp.float32)
        rows = []
        for bb in range(nb):
            mmat = _expand_rows(maskf[bb:bb + 1], h.shape[1])
            rows.append(jnp.max(h * mmat, axis=0, keepdims=True))
        acc_max[...] = jnp.maximum(acc_max[...],
                                   jnp.concatenate(rows, axis=0))

        @pl.when(i == nblk - 1)
        def _():
            z = jnp.dot(acc_add[...], w1a_ref[...],
                        preferred_element_type=jnp.float32)
            z += jnp.dot(acc_max[...], w1b_ref[...],
                         preferred_element_type=jnp.float32)
            z = jnp.maximum(z + b1_ref[...], 0.0)
            logits = jnp.dot(z, w2_ref[...],
                             preferred_element_type=jnp.float32) + b2_ref[...]
            m = jnp.max(logits, axis=-1, keepdims=True)
            e = jnp.exp(logits - m)
            out_ref[...] = e / jnp.sum(e, axis=-1, keepdims=True)

    return body


def kernel(x, edge_index, edge_weight, batch, W_in, b_in, W1, b1, W2, b2):
    n, d = x.shape
    h = W_in.shape[1]
    nb = 8
    cdim = W2.shape[1]
    e = edge_weight.shape[0]

    n_chunks = -(-e // (_NW * _LANES))
    ep = _NW * n_chunks * _LANES
    row = jnp.pad(edge_index[0], (0, ep - e)).reshape(_NW, n_chunks, _LANES)
    col = jnp.pad(edge_index[1], (0, ep - e)).reshape(_NW, n_chunks, _LANES)
    ew = jnp.pad(edge_weight, (0, ep - e)).reshape(_NW, n_chunks, _LANES)

    npt = -(-n // (16 * 8)) * 8
    degp = _make_deg_kernel(n_chunks, npt)(col, ew)

    r = 1000 if n % 1000 == 0 else n // 8
    nblk = n // r
    deg2 = degp.reshape(2, 16 * npt)[:, :n].reshape(2, nblk, 1, r)
    xw2, dinv = pl.pallas_call(
        _tc_xw2_body,
        grid=(nblk,),
        in_specs=[
            pl.BlockSpec((r, d), lambda i: (i, 0)),
            pl.BlockSpec((d, h), lambda i: (0, 0)),
            pl.BlockSpec((2, 1, 1, r), lambda i: (0, i, 0, 0)),
        ],
        out_specs=[
            pl.BlockSpec((r, h), lambda i: (i, 0)),
            pl.BlockSpec((r, 128), lambda i: (i, 0)),
        ],
        out_shape=[
            jax.ShapeDtypeStruct((n, h), jnp.float32),
            jax.ShapeDtypeStruct((n, 128), jnp.float32),
        ],
    )(x, W_in, deg2)

    n_pad = -(-n // (16 * 64)) * 16 * 64
    n_c = -(-e // (_NW * _CH))
    n_g = -(-n_c // (2 * _GC)) * 2
    n_c = n_g * _GC
    ep_c = _NW * n_c * _CH
    gidx = jnp.pad(edge_index[0], (0, ep_c - e)).reshape(
        _NW, n_g, _GC, _CH)
    cidx = jnp.pad(edge_index[1], (0, ep_c - e)).reshape(
        _NW, n_g, 2 * _GC, 64)
    ew_c = jnp.pad(edge_weight, (0, ep_c - e)).reshape(
        _NW, n_g, 8 * _GC, 16)
    xw2_bf = xw2.astype(jnp.bfloat16).reshape(n, 4, 2, 16)
    st = jnp.stack([xw2_bf[:, :, 0, :], xw2_bf[:, :, 1, :]], axis=-1)
    xw2p = jax.lax.bitcast_convert_type(st, jnp.int32).reshape(n, 64)
    pre = _make_edge_kernel(n_pad, n_g)(gidx, cidx, ew_c, xw2p)
    pre = pre.reshape(2, n_pad, 128)[:, :n]

    batch_r = batch.reshape(nblk, 1, r)
    w2p = jnp.zeros((h, 128), jnp.float32).at[:, :cdim].set(W2)
    b2p = jnp.full((1, 128), -1e30, jnp.float32).at[0, :cdim].set(b2)
    out = pl.pallas_call(
        _make_final_body(nblk, nb),
        grid=(nblk,),
        in_specs=[
            pl.BlockSpec((2, r, h), lambda i: (0, i, 0)),
            pl.BlockSpec((r, h), lambda i: (i, 0)),
            pl.BlockSpec((r, 128), lambda i: (i, 0)),
            pl.BlockSpec((1, 1, r), lambda i: (i, 0, 0)),
            pl.BlockSpec((1, h), lambda i: (0, 0)),
            pl.BlockSpec((h, h), lambda i: (0, 0)),
            pl.BlockSpec((h, h), lambda i: (0, 0)),
            pl.BlockSpec((1, h), lambda i: (0, 0)),
            pl.BlockSpec((h, 128), lambda i: (0, 0)),
            pl.BlockSpec((1, 128), lambda i: (0, 0)),
        ],
        out_specs=pl.BlockSpec((nb, 128), lambda i: (0, 0)),
        out_shape=jax.ShapeDtypeStruct((nb, 128), jnp.float32),
        scratch_shapes=[
            pltpu.VMEM((nb, h), jnp.float32),
            pltpu.VMEM((nb, h), jnp.float32),
        ],
        compiler_params=pltpu.CompilerParams(
            dimension_semantics=("arbitrary",)),
    )(pre, xw2, dinv, batch_r, b_in.reshape(1, h),
      W1[:h], W1[h:], b1.reshape(1, h), w2p, b2p)

    return out[:, :cdim]


# speedup vs baseline: 1.2218x; 1.2218x over previous
import functools
import math

import jax
import jax.numpy as jnp
from jax import lax
from jax.experimental import pallas as pl
from jax.experimental.pallas import tpu as pltpu
from jax.experimental.pallas import tpu_sc as plsc

_LANES = 128
_NW = 32

_GDN = lax.GatherDimensionNumbers(
    offset_dims=(), collapsed_slice_dims=(0,), start_index_map=(0,))


def _bcast_lane(vec, e):
    return lax.gather(vec, jnp.full((16, 1), e, jnp.int32), _GDN, (1,),
                      mode=lax.GatherScatterMode.PROMISE_IN_BOUNDS)


def _make_deg_kernel(n_chunks, npt):
    mesh = plsc.VectorSubcoreMesh(core_axis_name="c", subcore_axis_name="s")

    @functools.partial(
        pl.kernel,
        out_type=jax.ShapeDtypeStruct((2, 16, npt), jnp.float32),
        mesh=mesh,
        scratch_types=[
            pltpu.VMEM((n_chunks, _LANES), jnp.int32),
            pltpu.VMEM((n_chunks, _LANES), jnp.float32),
            pltpu.VMEM((npt,), jnp.float32),
            pltpu.VMEM_SHARED((16 * npt,), jnp.float32),
            pltpu.SemaphoreType.DMA,
        ],
    )
    def deg_kernel(col_hbm, ew_hbm, out_hbm, colv, ewv, zbuf, deg_sh, sem):
        c = lax.axis_index("c")
        s = lax.axis_index("s")
        wid = c * 16 + s

        def zero_body(i, carry):
            zbuf[pl.ds(i * 16, 16)] = jnp.zeros((16,), jnp.float32)
            return carry

        lax.fori_loop(0, npt // 16, zero_body, 0)

        pltpu.sync_copy(zbuf, deg_sh.at[pl.ds(s * npt, npt)])
        plsc.subcore_barrier()
        pltpu.sync_copy(col_hbm.at[wid], colv)
        pltpu.sync_copy(ew_hbm.at[wid], ewv)

        def body(j, carry):
            pltpu.sync_copy(ewv.at[j], deg_sh.at[colv.at[j]], add=True)
            return carry

        lax.fori_loop(0, n_chunks, body, 0)
        plsc.subcore_barrier()
        pltpu.sync_copy(deg_sh.at[pl.ds(s * npt, npt)], zbuf)
        pltpu.sync_copy(zbuf, out_hbm.at[c, s])

    return deg_kernel


_CH = 128


def _make_edge_kernel(n_pad, n_chunks):
    nps = n_pad // 16
    wb = _CH
    n_wb = nps // wb
    mesh = plsc.VectorSubcoreMesh(core_axis_name="c", subcore_axis_name="s")

    @functools.partial(
        pl.kernel,
        out_type=jax.ShapeDtypeStruct((2, 16, n_wb, wb, 128), jnp.float32),
        mesh=mesh,
        scratch_types=[
            pltpu.VMEM((2, _CH), jnp.int32),
            pltpu.VMEM((8, 16), jnp.float32),
            pltpu.VMEM((_CH, 128), jnp.float32),
            pltpu.VMEM_SHARED((n_pad, 128), jnp.float32),
            pltpu.SemaphoreType.DMA,
        ],
    )
    def edge_kernel(idx_hbm, ew_hbm, xw2_hbm, out_hbm,
                    idxv, ewv, buf, acc_sh, sem):
        c = lax.axis_index("c")
        s = lax.axis_index("s")
        wid = c * 16 + s

        def zero_body(t, carry):
            buf[t >> 3, pl.ds((t & 7) * 16, 16)] = jnp.zeros(
                (16,), jnp.float32)
            return carry

        lax.fori_loop(0, wb * 8, zero_body, 0)

        def zcopy(k, carry):
            pltpu.sync_copy(buf, acc_sh.at[pl.ds(s * nps + k * wb, wb)])
            return carry

        lax.fori_loop(0, n_wb, zcopy, 0)
        plsc.subcore_barrier()

        def chunk(j, carry):
            pltpu.sync_copy(idx_hbm.at[wid, j], idxv)
            pltpu.sync_copy(ew_hbm.at[wid, j], ewv)
            pltpu.async_copy(xw2_hbm.at[idxv.at[0]], buf, sem).wait()

            def scale(g, carry2):
                wvec = ewv[g, pl.ds(0, 16)]
                for e in range(16):
                    w = _bcast_lane(wvec, e)
                    rr = g * 16 + e
                    for f in range(8):
                        slc = pl.ds(f * 16, 16)
                        buf[rr, slc] = buf[rr, slc] * w
                return carry2

            lax.fori_loop(0, _CH // 16, scale, 0)
            pltpu.sync_copy(buf, acc_sh.at[idxv.at[1]], add=True)
            return carry

        lax.fori_loop(0, n_chunks, chunk, 0)
        plsc.subcore_barrier()

        def wback(k, carry):
            base = s * nps + k * wb
            pltpu.sync_copy(acc_sh.at[pl.ds(base, wb)], buf)
            pltpu.sync_copy(buf, out_hbm.at[c, s, k])
            return carry

        lax.fori_loop(0, n_wb, wback, 0)

    return edge_kernel


_OUTER_DN = (((0,), (0,)), ((), ()))


def _expand_rows(v_1r, width):
    ones = jnp.ones((1, width), jnp.float32)
    return lax.dot_general(v_1r, ones, _OUTER_DN,
                           preferred_element_type=jnp.float32)


def _tc_xw2_body(x_ref, w_ref, dm_ref, xw2_ref, dinv_ref):
    deg_row = dm_ref[0, 0] + dm_ref[1, 0] + 1.0
    deg = _expand_rows(deg_row, 128)
    dinv = lax.rsqrt(deg)
    xw = jnp.dot(x_ref[...], w_ref[...], preferred_element_type=jnp.float32)
    xw2_ref[...] = xw * dinv
    dinv_ref[...] = dinv


def _make_final_body(nblk, nb):
    def body(pre_ref, xw2_ref, dinv_ref, batch_ref, bin_ref,
             w1a_ref, w1b_ref, b1_ref, w2_ref, b2_ref, out_ref,
             acc_add, acc_max):
        i = pl.program_id(0)

        @pl.when(i == 0)
        def _():
            acc_add[...] = jnp.zeros_like(acc_add)
            acc_max[...] = jnp.zeros_like(acc_max)

        r = pre_ref.shape[1]
        pre = pre_ref[0] + pre_ref[1]
        h = jnp.maximum((pre + xw2_ref[...]) * dinv_ref[...] + bin_ref[...],
                        0.0)
        bv = batch_ref[0]
        mask = (jnp.broadcast_to(bv, (nb, r))
                == lax.broadcasted_iota(jnp.int32, (nb, r), 0))
        maskf = mask.astype(jnp.float32)
        acc_add[...] += jnp.dot(maskf, h,
                                preferred_element_type=jnp.float32)
        rows = []
        for bb in range(nb):
            mmat = _expand_rows(maskf[bb:bb + 1], h.shape[1])
            rows.append(jnp.max(h * mmat, axis=0, keepdims=True))
        acc_max[...] = jnp.maximum(acc_max[...],
                                   jnp.concatenate(rows, axis=0))

        @pl.when(i == nblk - 1)
        def _():
            z = jnp.dot(acc_add[...], w1a_ref[...],
                        preferred_element_type=jnp.float32)
            z += jnp.dot(acc_max[...], w1b_ref[...],
                         preferred_element_type=jnp.float32)
            z = jnp.maximum(z + b1_ref[...], 0.0)
            logits = jnp.dot(z, w2_ref[...],
                             preferred_element_type=jnp.float32) + b2_ref[...]
            m = jnp.max(logits, axis=-1, keepdims=True)
            e = jnp.exp(logits - m)
            out_ref[...] = e / jnp.sum(e, axis=-1, keepdims=True)

    return body


def kernel(x, edge_index, edge_weight, batch, W_in, b_in, W1, b1, W2, b2):
    n, d = x.shape
    h = W_in.shape[1]
    nb = 8
    cdim = W2.shape[1]
    e = edge_weight.shape[0]

    n_chunks = -(-e // (_NW * _LANES))
    ep = _NW * n_chunks * _LANES
    row = jnp.pad(edge_index[0], (0, ep - e)).reshape(_NW, n_chunks, _LANES)
    col = jnp.pad(edge_index[1], (0, ep - e)).reshape(_NW, n_chunks, _LANES)
    ew = jnp.pad(edge_weight, (0, ep - e)).reshape(_NW, n_chunks, _LANES)

    npt = -(-n // (16 * 8)) * 8
    degp = _make_deg_kernel(n_chunks, npt)(col, ew)

    r = 1000 if n % 1000 == 0 else n // 8
    nblk = n // r
    deg2 = degp.reshape(2, 16 * npt)[:, :n].reshape(2, nblk, 1, r)
    xw2, dinv = pl.pallas_call(
        _tc_xw2_body,
        grid=(nblk,),
        in_specs=[
            pl.BlockSpec((r, d), lambda i: (i, 0)),
            pl.BlockSpec((d, h), lambda i: (0, 0)),
            pl.BlockSpec((2, 1, 1, r), lambda i: (0, i, 0, 0)),
        ],
        out_specs=[
            pl.BlockSpec((r, h), lambda i: (i, 0)),
            pl.BlockSpec((r, 128), lambda i: (i, 0)),
        ],
        out_shape=[
            jax.ShapeDtypeStruct((n, h), jnp.float32),
            jax.ShapeDtypeStruct((n, 128), jnp.float32),
        ],
    )(x, W_in, deg2)

    n_pad = -(-n // (16 * _CH)) * 16 * _CH
    n_c = -(-e // (_NW * _CH))
    ep_c = _NW * n_c * _CH
    row_c = jnp.pad(edge_index[0], (0, ep_c - e)).reshape(_NW, n_c, _CH)
    col_c = jnp.pad(edge_index[1], (0, ep_c - e)).reshape(_NW, n_c, _CH)
    idx_c = jnp.stack([row_c, col_c], axis=2)
    ew_c = jnp.pad(edge_weight, (0, ep_c - e)).reshape(_NW, n_c, 8, 16)
    pre = _make_edge_kernel(n_pad, n_c)(idx_c, ew_c, xw2)
    pre = pre.reshape(2, n_pad, 128)[:, :n]

    batch_r = batch.reshape(nblk, 1, r)
    w2p = jnp.zeros((h, 128), jnp.float32).at[:, :cdim].set(W2)
    b2p = jnp.full((1, 128), -1e30, jnp.float32).at[0, :cdim].set(b2)
    out = pl.pallas_call(
        _make_final_body(nblk, nb),
        grid=(nblk,),
        in_specs=[
            pl.BlockSpec((2, r, h), lambda i: (0, i, 0)),
            pl.BlockSpec((r, h), lambda i: (i, 0)),
            pl.BlockSpec((r, 128), lambda i: (i, 0)),
            pl.BlockSpec((1, 1, r), lambda i: (i, 0, 0)),
            pl.BlockSpec((1, h), lambda i: (0, 0)),
            pl.BlockSpec((h, h), lambda i: (0, 0)),
            pl.BlockSpec((h, h), lambda i: (0, 0)),
            pl.BlockSpec((1, h), lambda i: (0, 0)),
            pl.BlockSpec((h, 128), lambda i: (0, 0)),
            pl.BlockSpec((1, 128), lambda i: (0, 0)),
        ],
        out_specs=pl.BlockSpec((nb, 128), lambda i: (0, 0)),
        out_shape=jax.ShapeDtypeStruct((nb, 128), jnp.float32),
        scratch_shapes=[
            pltpu.VMEM((nb, h), jnp.float32),
            pltpu.VMEM((nb, h), jnp.float32),
        ],
        compiler_params=pltpu.CompilerParams(
            dimension_semantics=("arbitrary",)),
    )(pre, xw2, dinv, batch_r, b_in.reshape(1, h),
      W1[:h], W1[h:], b1.reshape(1, h), w2p, b2p)

    return out[:, :cdim]
